# Initial kernel scaffold; baseline (speedup 1.0000x reference)
import jax
import jax.numpy as jnp
from jax.experimental import pallas as pl


def kernel(x, edge_index, W_gat, att_src, att_dst, b_gat, W_gcn, b_gcn, W_sage_l, W_sage_r, b_sage, gamma, beta, W1, b1, W2, b2, W3, b3):
    raise NotImplementedError("write your pallas kernel here")


# speedup vs baseline: 10.1172x; 10.1172x over previous
import dataclasses
import functools

import jax
import jax.numpy as jnp
from jax import lax
from jax.experimental import pallas as pl
from jax.experimental.pallas import tpu as pltpu
from jax.experimental.pallas import tpu_sc as plsc

N = 10000
E = 320000
D = 128
H = 128
OUT = 128

NC = 2
NS = 16
GW = 80
NG = 4096
EP = NG * GW
GPT = NG // (NC * NS)
GPC = NG // NS
NPAD = 10240
RB = 1000
HC = 64


def _sc_compiler_params():
    cp = pltpu.CompilerParams()
    if "needs_layout_passes" in pltpu.CompilerParams.__dataclass_fields__:
        cp = dataclasses.replace(cp, needs_layout_passes=False)
    return cp


def _prep_body(x_ref, wg_ref, wc_ref, as_ref, ad_ref,
               h_ref, hg_ref, u_ref, v_ref, mx_ref):
    xb = x_ref[...]
    hb = jnp.dot(xb, wg_ref[...], preferred_element_type=jnp.float32)
    h_ref[...] = hb
    hg_ref[...] = jnp.dot(xb, wc_ref[...], preferred_element_type=jnp.float32)
    ub = jnp.sum(hb * as_ref[...], axis=1)
    vb = jnp.sum(hb * ad_ref[...], axis=1)
    u_ref[...] = ub[:, None]
    v_ref[...] = vb[:, None]

    @pl.when(pl.program_id(0) == 0)
    def _():
        mx_ref[...] = jnp.full((1, 1), -1e30, jnp.float32)

    mx_ref[...] = jnp.maximum(mx_ref[...], jnp.max(ub))


def _tc_prep(x, W_gat, W_gcn, att_src, att_dst):
    return pl.pallas_call(
        _prep_body,
        grid=(N // RB,),
        in_specs=[
            pl.BlockSpec((RB, D), lambda i: (i, 0)),
            pl.BlockSpec((D, H), lambda i: (0, 0)),
            pl.BlockSpec((D, H), lambda i: (0, 0)),
            pl.BlockSpec((1, H), lambda i: (0, 0)),
            pl.BlockSpec((1, H), lambda i: (0, 0)),
        ],
        out_specs=[
            pl.BlockSpec((RB, H), lambda i: (i, 0)),
            pl.BlockSpec((RB, H), lambda i: (i, 0)),
            pl.BlockSpec((RB, 1), lambda i: (i, 0)),
            pl.BlockSpec((RB, 1), lambda i: (i, 0)),
            pl.BlockSpec((1, 1), lambda i: (0, 0)),
        ],
        out_shape=[
            jax.ShapeDtypeStruct((N, H), jnp.float32),
            jax.ShapeDtypeStruct((N, H), jnp.float32),
            jax.ShapeDtypeStruct((N, 1), jnp.float32),
            jax.ShapeDtypeStruct((N, 1), jnp.float32),
            jax.ShapeDtypeStruct((1, 1), jnp.float32),
        ],
    )(x, W_gat, W_gcn, att_src.reshape(1, H), att_dst.reshape(1, H))


def _sc_scalar_body(src_hbm, dst_hbm, u_hbm, v_hbm, mx_hbm,
                    ex_hbm, den_hbm, cnt_hbm,
                    u_tab, v_tab, mx_tab, sbuf, dbuf, exbuf, ones, zbuf,
                    den_sp, cnt_sp):
    cid = lax.axis_index("c")
    sid = lax.axis_index("s")
    wid = sid * NC + cid

    @pl.loop(0, 128)
    def _(i):
        zbuf[pl.ds(i * 16, 16)] = jnp.zeros((16,), jnp.float32)

    @pl.when(sid == 0)
    def _():
        @pl.loop(0, 5)
        def _(k):
            pltpu.sync_copy(zbuf, den_sp.at[pl.ds(k * 2048, 2048)])
            pltpu.sync_copy(zbuf, cnt_sp.at[pl.ds(k * 2048, 2048)])

    @pl.loop(0, 5)
    def _(i):
        ones[pl.ds(i * 16, 16)] = jnp.full((16,), 1.0, jnp.float32)

    pltpu.sync_copy(u_hbm, u_tab)
    pltpu.sync_copy(v_hbm, v_tab)
    pltpu.sync_copy(mx_hbm, mx_tab)
    plsc.subcore_barrier()

    mx = mx_tab[...]
    row0 = wid * GPT

    @pl.loop(0, 8)
    def _(c):
        r = row0 + c * 16
        pltpu.sync_copy(src_hbm.at[pl.ds(r, 16)], sbuf)
        pltpu.sync_copy(dst_hbm.at[pl.ds(r, 16)], dbuf)

        @pl.loop(0, 16)
        def _(j):
            @pl.loop(0, 5)
            def _(i):
                s16 = sbuf[j, pl.ds(i * 16, 16)]
                d16 = dbuf[j, pl.ds(i * 16, 16)]
                ug = plsc.load_gather(u_tab, [s16])
                vg = plsc.load_gather(v_tab, [d16])
                e = ug + vg
                e = jnp.where(e > 0, e, 0.2 * e)
                mp = mx + vg
                mp = jnp.where(mp > 0, mp, 0.2 * mp)
                exbuf[j, pl.ds(i * 16, 16)] = jnp.exp(e - mp)

        pltpu.sync_copy(exbuf, ex_hbm.at[pl.ds(r, 16)])

        @pl.loop(0, 16)
        def _(j):
            pltpu.sync_copy(exbuf.at[j], den_sp.at[dbuf.at[j]], add=True)
            pltpu.sync_copy(ones, cnt_sp.at[dbuf.at[j]], add=True)

    plsc.subcore_barrier()

    @pl.when(sid == 0)
    def _():
        pltpu.sync_copy(den_sp.at[pl.ds(0, NPAD)], den_hbm.at[cid])
        pltpu.sync_copy(cnt_sp.at[pl.ds(0, NPAD)], cnt_hbm.at[cid])


def _sc_scalar(src2d, dst2d, u, v, mx16):
    mesh = plsc.VectorSubcoreMesh(core_axis_name="c", subcore_axis_name="s")
    kern = pl.kernel(
        _sc_scalar_body,
        mesh=mesh,
        out_type=(
            jax.ShapeDtypeStruct((NG, GW), jnp.float32),
            jax.ShapeDtypeStruct((NC, NPAD), jnp.float32),
            jax.ShapeDtypeStruct((NC, NPAD), jnp.float32),
        ),
        scratch_types=[
            pltpu.VMEM((N,), jnp.float32),
            pltpu.VMEM((N,), jnp.float32),
            pltpu.VMEM((16,), jnp.float32),
            pltpu.VMEM((16, GW), jnp.int32),
            pltpu.VMEM((16, GW), jnp.int32),
            pltpu.VMEM((16, GW), jnp.float32),
            pltpu.VMEM((GW,), jnp.float32),
            pltpu.VMEM((2048,), jnp.float32),
            pltpu.VMEM_SHARED((NPAD,), jnp.float32),
            pltpu.VMEM_SHARED((NPAD,), jnp.float32),
        ],
        compiler_params=_sc_compiler_params(),
    )
    return kern(src2d, dst2d, u, v, mx16)


def _mid_body(h_ref, hg_ref, x_ref, cp_ref, dp_ref,
              g_ref, cnt_ref, den_ref):
    cnt = cp_ref[...][:, 0] + cp_ref[...][:, 1]
    den = dp_ref[...][:, 0] + dp_ref[...][:, 1]
    cnt_ref[...] = cnt[:, None]
    den_ref[...] = den[:, None]
    dinv = lax.rsqrt(cnt + 1.0)
    g_ref[0] = h_ref[...]
    g_ref[1] = hg_ref[...] * dinv[:, None]
    g_ref[2] = x_ref[...]


def _tc_mid(h, hg, x, cnt_part, den_part):
    return pl.pallas_call(
        _mid_body,
        grid=(N // RB,),
        in_specs=[
            pl.BlockSpec((RB, H), lambda i: (i, 0)),
            pl.BlockSpec((RB, H), lambda i: (i, 0)),
            pl.BlockSpec((RB, D), lambda i: (i, 0)),
            pl.BlockSpec((RB, NC), lambda i: (i, 0)),
            pl.BlockSpec((RB, NC), lambda i: (i, 0)),
        ],
        out_specs=[
            pl.BlockSpec((3, RB, H), lambda i: (0, i, 0)),
            pl.BlockSpec((RB, 1), lambda i: (i, 0)),
            pl.BlockSpec((RB, 1), lambda i: (i, 0)),
        ],
        out_shape=[
            jax.ShapeDtypeStruct((3, N, H), jnp.float32),
            jax.ShapeDtypeStruct((N, 1), jnp.float32),
            jax.ShapeDtypeStruct((N, 1), jnp.float32),
        ],
    )(h, hg, x, cnt_part, den_part)


def _sc_feat_body(g_hbm, src_hbm, dst_hbm, ex_hbm,
                  s_hbm,
                  rows, sbuf, dbuf, exbuf, zbuf, acc):
    cid = lax.axis_index("c")
    sid = lax.axis_index("s")
    wid = sid * NC + cid
    row0 = wid * GPT

    @pl.loop(0, 128)
    def _(i):
        @pl.loop(0, 8)
        def _(k):
            zbuf[i, pl.ds(k * 16, 16)] = jnp.zeros((16,), jnp.float32)

    for f in range(3):
        gq = g_hbm.at[f]
        sq = s_hbm.at[NC * f + cid]

        @pl.loop(0, 5)
        def _(k):
            pltpu.sync_copy(zbuf, acc.at[pl.ds(sid * 640 + k * 128, 128)])

        plsc.subcore_barrier()

        @pl.loop(0, 16)
        def _(c):
            r = row0 + c * 8
            pltpu.sync_copy(src_hbm.at[pl.ds(r, 8)], sbuf)
            pltpu.sync_copy(dst_hbm.at[pl.ds(r, 8)], dbuf)
            if f == 0:
                pltpu.sync_copy(ex_hbm.at[pl.ds(r, 8)], exbuf)

            @pl.loop(0, 8)
            def _(j):
                pltpu.sync_copy(gq.at[sbuf.at[j]], rows)
                if f == 0:
                    jb = jnp.broadcast_to(j, (16,))

                    @pl.loop(0, GW)
                    def _(t):
                        exv = plsc.load_gather(
                            exbuf, [jb, jnp.broadcast_to(t, (16,))])
                        for k in range(H // 16):
                            rows[t, pl.ds(k * 16, 16)] = (
                                rows[t, pl.ds(k * 16, 16)] * exv)

                pltpu.sync_copy(rows, acc.at[dbuf.at[j]], add=True)

        plsc.subcore_barrier()
        pltpu.sync_copy(acc.at[pl.ds(sid * 640, 640)],
                        sq.at[pl.ds(sid * 640, 640)])
        plsc.subcore_barrier()


def _sc_feat(G, src2d, dst2d, ex2d):
    mesh = plsc.VectorSubcoreMesh(core_axis_name="c", subcore_axis_name="s")
    kern = pl.kernel(
        _sc_feat_body,
        mesh=mesh,
        out_type=jax.ShapeDtypeStruct((3 * NC, NPAD, H), jnp.float32),
        scratch_types=[
            pltpu.VMEM((GW, H), jnp.float32),
            pltpu.VMEM((8, GW), jnp.int32),
            pltpu.VMEM((8, GW), jnp.int32),
            pltpu.VMEM((8, GW), jnp.float32),
            pltpu.VMEM((128, H), jnp.float32),
            pltpu.VMEM_SHARED((NPAD, H), jnp.float32),
        ],
        compiler_params=_sc_compiler_params(),
    )
    return kern(G, src2d, dst2d, ex2d)


def _final_body(s_ref, h_ref, hg_ref, x_ref, u_ref, v_ref, mx_ref,
                cnt_ref, den_ref,
                bgat_ref, bgcn_ref, wsl_ref, wsr_ref, bsage_ref,
                gamma_ref, beta_ref, w1_ref, b1_ref, w2_ref, b2_ref,
                w3_ref, b3_ref, out_ref):
    sgat = s_ref[0] + s_ref[1]
    sgcn = s_ref[2] + s_ref[3]
    ssag = s_ref[4] + s_ref[5]

    u = u_ref[...][:, 0]
    v = v_ref[...][:, 0]
    mx = mx_ref[...][0, 0]
    es = u + v
    es = jnp.where(es > 0, es, 0.2 * es)
    mp = mx + v
    mp = jnp.where(mp > 0, mp, 0.2 * mp)
    exs = jnp.exp(es - mp)

    den = den_ref[...][:, 0] + exs
    r = 1.0 / (den + 1e-16)
    x_gat = (sgat + exs[:, None] * h_ref[...]) * r[:, None] + bgat_ref[...]

    cnt = cnt_ref[...][:, 0]
    dinv = lax.rsqrt(cnt + 1.0)
    x_gcn = (dinv[:, None] * sgcn + (dinv * dinv)[:, None] * hg_ref[...]
             + bgcn_ref[...])

    agg = ssag / jnp.maximum(cnt, 1.0)[:, None]
    x_sage = (jnp.dot(agg, wsl_ref[...], preferred_element_type=jnp.float32)
              + bsage_ref[...]
              + jnp.dot(x_ref[...], wsr_ref[...],
                        preferred_element_type=jnp.float32))

    cat = jnp.concatenate([x_gat, x_gcn, x_sage], axis=1)
    bn = cat * gamma_ref[...] + beta_ref[...]
    h1 = jnp.dot(jnp.maximum(bn, 0.0), w1_ref[...],
                 preferred_element_type=jnp.float32) + b1_ref[...]
    h2 = jnp.dot(jnp.maximum(h1, 0.0), w2_ref[...],
                 preferred_element_type=jnp.float32) + b2_ref[...]
    out_ref[...] = jnp.dot(jnp.maximum(h2, 0.0), w3_ref[...],
                           preferred_element_type=jnp.float32) + b3_ref[...]


def _tc_final(S, h, hg, x, u, v, mx, cnt, den_e,
              b_gat, b_gcn, W_sage_l, W_sage_r, b_sage, gamma_s, beta,
              W1, b1, W2, b2, W3, b3):
    row = lambda i: (i, 0)
    full2 = lambda i: (0, 0)
    return pl.pallas_call(
        _final_body,
        grid=(N // RB,),
        in_specs=[
            pl.BlockSpec((3 * NC, RB, H), lambda i: (0, i, 0)),
            pl.BlockSpec((RB, H), row),
            pl.BlockSpec((RB, H), row),
            pl.BlockSpec((RB, D), row),
            pl.BlockSpec((RB, 1), row),
            pl.BlockSpec((RB, 1), row),
            pl.BlockSpec((1, 1), full2),
            pl.BlockSpec((RB, 1), row),
            pl.BlockSpec((RB, 1), row),
            pl.BlockSpec((1, H), full2),
            pl.BlockSpec((1, H), full2),
            pl.BlockSpec((D, H), full2),
            pl.BlockSpec((D, H), full2),
            pl.BlockSpec((1, H), full2),
            pl.BlockSpec((1, 3 * H), full2),
            pl.BlockSpec((1, 3 * H), full2),
            pl.BlockSpec((3 * H, 2 * H), full2),
            pl.BlockSpec((1, 2 * H), full2),
            pl.BlockSpec((2 * H, H), full2),
            pl.BlockSpec((1, H), full2),
            pl.BlockSpec((H, OUT), full2),
            pl.BlockSpec((1, OUT), full2),
        ],
        out_specs=pl.BlockSpec((RB, OUT), row),
        out_shape=jax.ShapeDtypeStruct((N, OUT), jnp.float32),
    )(S, h, hg, x, u, v, mx, cnt, den_e,
      b_gat.reshape(1, H), b_gcn.reshape(1, H), W_sage_l, W_sage_r,
      b_sage.reshape(1, H), gamma_s.reshape(1, 3 * H), beta.reshape(1, 3 * H),
      W1, b1.reshape(1, 2 * H), W2, b2.reshape(1, H), W3, b3.reshape(1, OUT))


def kernel(x, edge_index, W_gat, att_src, att_dst, b_gat, W_gcn, b_gcn,
           W_sage_l, W_sage_r, b_sage, gamma, beta, W1, b1, W2, b2, W3, b3):
    npad = EP - E
    pad_src = jnp.zeros((npad,), jnp.int32)
    pad_dst = N + (jnp.arange(npad, dtype=jnp.int32) % 8)
    src2d = jnp.concatenate([edge_index[0], pad_src]).reshape(NG, GW)
    dst2d = jnp.concatenate([edge_index[1], pad_dst]).reshape(NG, GW)

    h, hg, u, v, mx = _tc_prep(x, W_gat, W_gcn, att_src, att_dst)
    mx16 = jnp.broadcast_to(mx.reshape(1), (16,))

    ex2d, den_part, cnt_part = _sc_scalar(src2d, dst2d,
                                          u.reshape(N), v.reshape(N), mx16)
    G, cnt, den_e = _tc_mid(h, hg, x, cnt_part.T[:N], den_part.T[:N])
    S = _sc_feat(G, src2d, dst2d, ex2d)

    gamma_s = gamma * (1.0 / jnp.sqrt(1.0 + 1e-5))
    return _tc_final(S, h, hg, x, u, v, mx, cnt, den_e,
                     b_gat, b_gcn, W_sage_l, W_sage_r, b_sage, gamma_s, beta,
                     W1, b1, W2, b2, W3, b3)


# speedup vs baseline: 10.8796x; 1.0754x over previous
import dataclasses
import functools

import jax
import jax.numpy as jnp
from jax import lax
from jax.experimental import pallas as pl
from jax.experimental.pallas import tpu as pltpu
from jax.experimental.pallas import tpu_sc as plsc

N = 10000
E = 320000
D = 128
H = 128
OUT = 128

NC = 2
NS = 16
GW = 128
NG = 2560
EP = NG * GW
GPT = NG // (NC * NS)
NPAD = 10240
RB = 1000
HC = 64


def _sc_compiler_params():
    cp = pltpu.CompilerParams()
    if "needs_layout_passes" in pltpu.CompilerParams.__dataclass_fields__:
        cp = dataclasses.replace(cp, needs_layout_passes=False)
    return cp


def _prep_body(x_ref, wg_ref, wc_ref, as_ref, ad_ref,
               h_ref, hg_ref, u_ref, v_ref, mx_ref):
    xb = x_ref[...]
    hb = jnp.dot(xb, wg_ref[...], preferred_element_type=jnp.float32)
    h_ref[...] = hb
    hg_ref[...] = jnp.dot(xb, wc_ref[...], preferred_element_type=jnp.float32)
    ub = jnp.sum(hb * as_ref[...], axis=1)
    vb = jnp.sum(hb * ad_ref[...], axis=1)
    u_ref[...] = ub[:, None]
    v_ref[...] = vb[:, None]

    @pl.when(pl.program_id(0) == 0)
    def _():
        mx_ref[...] = jnp.full((1, 1), -1e30, jnp.float32)

    mx_ref[...] = jnp.maximum(mx_ref[...], jnp.max(ub))


def _tc_prep(x, W_gat, W_gcn, att_src, att_dst):
    return pl.pallas_call(
        _prep_body,
        grid=(N // RB,),
        in_specs=[
            pl.BlockSpec((RB, D), lambda i: (i, 0)),
            pl.BlockSpec((D, H), lambda i: (0, 0)),
            pl.BlockSpec((D, H), lambda i: (0, 0)),
            pl.BlockSpec((1, H), lambda i: (0, 0)),
            pl.BlockSpec((1, H), lambda i: (0, 0)),
        ],
        out_specs=[
            pl.BlockSpec((RB, H), lambda i: (i, 0)),
            pl.BlockSpec((RB, H), lambda i: (i, 0)),
            pl.BlockSpec((RB, 1), lambda i: (i, 0)),
            pl.BlockSpec((RB, 1), lambda i: (i, 0)),
            pl.BlockSpec((1, 1), lambda i: (0, 0)),
        ],
        out_shape=[
            jax.ShapeDtypeStruct((N, H), jnp.float32),
            jax.ShapeDtypeStruct((N, H), jnp.float32),
            jax.ShapeDtypeStruct((N, 1), jnp.float32),
            jax.ShapeDtypeStruct((N, 1), jnp.float32),
            jax.ShapeDtypeStruct((1, 1), jnp.float32),
        ],
    )(x, W_gat, W_gcn, att_src.reshape(1, H), att_dst.reshape(1, H))


def _sc_scalar_body(src_hbm, dst_hbm, u_hbm, v_hbm, mx_hbm,
                    ex_hbm, den_hbm, cnt_hbm,
                    u_tab, v_tab, mx_tab, sbuf, dbuf, exbuf, ones, zbuf,
                    den_sp, cnt_sp):
    cid = lax.axis_index("c")
    sid = lax.axis_index("s")
    wid = sid * NC + cid

    @pl.loop(0, 128)
    def _(i):
        zbuf[pl.ds(i * 16, 16)] = jnp.zeros((16,), jnp.float32)

    @pl.when(sid == 0)
    def _():
        @pl.loop(0, 5)
        def _(k):
            pltpu.sync_copy(zbuf, den_sp.at[pl.ds(k * 2048, 2048)])
            pltpu.sync_copy(zbuf, cnt_sp.at[pl.ds(k * 2048, 2048)])

    @pl.loop(0, GW // 16)
    def _(i):
        ones[pl.ds(i * 16, 16)] = jnp.full((16,), 1.0, jnp.float32)

    pltpu.sync_copy(u_hbm, u_tab)
    pltpu.sync_copy(v_hbm, v_tab)
    pltpu.sync_copy(mx_hbm, mx_tab)
    plsc.subcore_barrier()

    mx = mx_tab[...]
    row0 = wid * GPT

    @pl.loop(0, 10)
    def _(c):
        r = row0 + c * 8
        pltpu.sync_copy(src_hbm.at[pl.ds(r, 8)], sbuf)
        pltpu.sync_copy(dst_hbm.at[pl.ds(r, 8)], dbuf)

        @pl.loop(0, 8)
        def _(j):
            @pl.loop(0, GW // 16)
            def _(i):
                s16 = sbuf[j, pl.ds(i * 16, 16)]
                d16 = dbuf[j, pl.ds(i * 16, 16)]
                ug = plsc.load_gather(u_tab, [s16])
                vg = plsc.load_gather(v_tab, [d16])
                e = ug + vg
                e = jnp.where(e > 0, e, 0.2 * e)
                mp = mx + vg
                mp = jnp.where(mp > 0, mp, 0.2 * mp)
                exbuf[j, pl.ds(i * 16, 16)] = jnp.exp(e - mp)

        pltpu.sync_copy(exbuf, ex_hbm.at[pl.ds(r, 8)])

        @pl.loop(0, 8)
        def _(j):
            pltpu.sync_copy(exbuf.at[j], den_sp.at[dbuf.at[j]], add=True)
            pltpu.sync_copy(ones, cnt_sp.at[dbuf.at[j]], add=True)

    plsc.subcore_barrier()

    @pl.when(sid == 0)
    def _():
        pltpu.sync_copy(den_sp.at[pl.ds(0, NPAD)], den_hbm.at[cid])
        pltpu.sync_copy(cnt_sp.at[pl.ds(0, NPAD)], cnt_hbm.at[cid])


def _sc_scalar(src2d, dst2d, u, v, mx16):
    mesh = plsc.VectorSubcoreMesh(core_axis_name="c", subcore_axis_name="s")
    kern = pl.kernel(
        _sc_scalar_body,
        mesh=mesh,
        out_type=(
            jax.ShapeDtypeStruct((NG, GW), jnp.float32),
            jax.ShapeDtypeStruct((NC, NPAD), jnp.float32),
            jax.ShapeDtypeStruct((NC, NPAD), jnp.float32),
        ),
        scratch_types=[
            pltpu.VMEM((N,), jnp.float32),
            pltpu.VMEM((N,), jnp.float32),
            pltpu.VMEM((16,), jnp.float32),
            pltpu.VMEM((8, GW), jnp.int32),
            pltpu.VMEM((8, GW), jnp.int32),
            pltpu.VMEM((8, GW), jnp.float32),
            pltpu.VMEM((GW,), jnp.float32),
            pltpu.VMEM((2048,), jnp.float32),
            pltpu.VMEM_SHARED((NPAD,), jnp.float32),
            pltpu.VMEM_SHARED((NPAD,), jnp.float32),
        ],
        compiler_params=_sc_compiler_params(),
    )
    return kern(src2d, dst2d, u, v, mx16)


def _mid_body(h_ref, hg_ref, x_ref, cp_ref, dp_ref,
              g_ref, cnt_ref, den_ref):
    cnt = cp_ref[...][:, 0] + cp_ref[...][:, 1]
    den = dp_ref[...][:, 0] + dp_ref[...][:, 1]
    cnt_ref[...] = cnt[:, None]
    den_ref[...] = den[:, None]
    dinv = lax.rsqrt(cnt + 1.0)
    g_ref[0] = h_ref[...]
    g_ref[1] = hg_ref[...] * dinv[:, None]
    g_ref[2] = x_ref[...]


def _tc_mid(h, hg, x, cnt_part, den_part):
    return pl.pallas_call(
        _mid_body,
        grid=(N // RB,),
        in_specs=[
            pl.BlockSpec((RB, H), lambda i: (i, 0)),
            pl.BlockSpec((RB, H), lambda i: (i, 0)),
            pl.BlockSpec((RB, D), lambda i: (i, 0)),
            pl.BlockSpec((RB, NC), lambda i: (i, 0)),
            pl.BlockSpec((RB, NC), lambda i: (i, 0)),
        ],
        out_specs=[
            pl.BlockSpec((3, RB, H), lambda i: (0, i, 0)),
            pl.BlockSpec((RB, 1), lambda i: (i, 0)),
            pl.BlockSpec((RB, 1), lambda i: (i, 0)),
        ],
        out_shape=[
            jax.ShapeDtypeStruct((3, N, H), jnp.float32),
            jax.ShapeDtypeStruct((N, 1), jnp.float32),
            jax.ShapeDtypeStruct((N, 1), jnp.float32),
        ],
    )(h, hg, x, cnt_part, den_part)


def _sc_feat_body(g_hbm, src_hbm, dst_hbm, ex_hbm,
                  s_hbm,
                  rows, sbuf, dbuf, exbuf, zbuf, acc):
    cid = lax.axis_index("c")
    sid = lax.axis_index("s")
    wid = sid * NC + cid
    row0 = wid * GPT

    @pl.loop(0, 128)
    def _(i):
        @pl.loop(0, 8)
        def _(k):
            zbuf[i, pl.ds(k * 16, 16)] = jnp.zeros((16,), jnp.float32)

    for f in range(3):
        gq = g_hbm.at[f]
        sq = s_hbm.at[NC * f + cid]

        @pl.loop(0, 5)
        def _(k):
            pltpu.sync_copy(zbuf, acc.at[pl.ds(sid * 640 + k * 128, 128)])

        plsc.subcore_barrier()

        @pl.loop(0, 10)
        def _(c):
            r = row0 + c * 8
            pltpu.sync_copy(src_hbm.at[pl.ds(r, 8)], sbuf)
            pltpu.sync_copy(dst_hbm.at[pl.ds(r, 8)], dbuf)
            if f == 0:
                pltpu.sync_copy(ex_hbm.at[pl.ds(r, 8)], exbuf)

            @pl.loop(0, 8)
            def _(j):
                pltpu.sync_copy(gq.at[sbuf.at[j]], rows)
                if f == 0:
                    jb = jnp.broadcast_to(j, (16,))

                    @pl.loop(0, GW)
                    def _(t):
                        exv = plsc.load_gather(
                            exbuf, [jb, jnp.broadcast_to(t, (16,))])
                        for k in range(H // 16):
                            rows[t, pl.ds(k * 16, 16)] = (
                                rows[t, pl.ds(k * 16, 16)] * exv)

                pltpu.sync_copy(rows, acc.at[dbuf.at[j]], add=True)

        plsc.subcore_barrier()
        pltpu.sync_copy(acc.at[pl.ds(sid * 640, 640)],
                        sq.at[pl.ds(sid * 640, 640)])
        plsc.subcore_barrier()


def _sc_feat(G, src2d, dst2d, ex2d):
    mesh = plsc.VectorSubcoreMesh(core_axis_name="c", subcore_axis_name="s")
    kern = pl.kernel(
        _sc_feat_body,
        mesh=mesh,
        out_type=jax.ShapeDtypeStruct((3 * NC, NPAD, H), jnp.float32),
        scratch_types=[
            pltpu.VMEM((GW, H), jnp.float32),
            pltpu.VMEM((8, GW), jnp.int32),
            pltpu.VMEM((8, GW), jnp.int32),
            pltpu.VMEM((8, GW), jnp.float32),
            pltpu.VMEM((128, H), jnp.float32),
            pltpu.VMEM_SHARED((NPAD, H), jnp.float32),
        ],
        compiler_params=_sc_compiler_params(),
    )
    return kern(G, src2d, dst2d, ex2d)


def _final_body(s_ref, h_ref, hg_ref, x_ref, u_ref, v_ref, mx_ref,
                cnt_ref, den_ref,
                bgat_ref, bgcn_ref, wsl_ref, wsr_ref, bsage_ref,
                gamma_ref, beta_ref, w1_ref, b1_ref, w2_ref, b2_ref,
                w3_ref, b3_ref, out_ref):
    sgat = s_ref[0] + s_ref[1]
    sgcn = s_ref[2] + s_ref[3]
    ssag = s_ref[4] + s_ref[5]

    u = u_ref[...][:, 0]
    v = v_ref[...][:, 0]
    mx = mx_ref[...][0, 0]
    es = u + v
    es = jnp.where(es > 0, es, 0.2 * es)
    mp = mx + v
    mp = jnp.where(mp > 0, mp, 0.2 * mp)
    exs = jnp.exp(es - mp)

    den = den_ref[...][:, 0] + exs
    r = 1.0 / (den + 1e-16)
    x_gat = (sgat + exs[:, None] * h_ref[...]) * r[:, None] + bgat_ref[...]

    cnt = cnt_ref[...][:, 0]
    dinv = lax.rsqrt(cnt + 1.0)
    x_gcn = (dinv[:, None] * sgcn + (dinv * dinv)[:, None] * hg_ref[...]
             + bgcn_ref[...])

    agg = ssag / jnp.maximum(cnt, 1.0)[:, None]
    x_sage = (jnp.dot(agg, wsl_ref[...], preferred_element_type=jnp.float32)
              + bsage_ref[...]
              + jnp.dot(x_ref[...], wsr_ref[...],
                        preferred_element_type=jnp.float32))

    cat = jnp.concatenate([x_gat, x_gcn, x_sage], axis=1)
    bn = cat * gamma_ref[...] + beta_ref[...]
    h1 = jnp.dot(jnp.maximum(bn, 0.0), w1_ref[...],
                 preferred_element_type=jnp.float32) + b1_ref[...]
    h2 = jnp.dot(jnp.maximum(h1, 0.0), w2_ref[...],
                 preferred_element_type=jnp.float32) + b2_ref[...]
    out_ref[...] = jnp.dot(jnp.maximum(h2, 0.0), w3_ref[...],
                           preferred_element_type=jnp.float32) + b3_ref[...]


def _tc_final(S, h, hg, x, u, v, mx, cnt, den_e,
              b_gat, b_gcn, W_sage_l, W_sage_r, b_sage, gamma_s, beta,
              W1, b1, W2, b2, W3, b3):
    row = lambda i: (i, 0)
    full2 = lambda i: (0, 0)
    return pl.pallas_call(
        _final_body,
        grid=(N // RB,),
        in_specs=[
            pl.BlockSpec((3 * NC, RB, H), lambda i: (0, i, 0)),
            pl.BlockSpec((RB, H), row),
            pl.BlockSpec((RB, H), row),
            pl.BlockSpec((RB, D), row),
            pl.BlockSpec((RB, 1), row),
            pl.BlockSpec((RB, 1), row),
            pl.BlockSpec((1, 1), full2),
            pl.BlockSpec((RB, 1), row),
            pl.BlockSpec((RB, 1), row),
            pl.BlockSpec((1, H), full2),
            pl.BlockSpec((1, H), full2),
            pl.BlockSpec((D, H), full2),
            pl.BlockSpec((D, H), full2),
            pl.BlockSpec((1, H), full2),
            pl.BlockSpec((1, 3 * H), full2),
            pl.BlockSpec((1, 3 * H), full2),
            pl.BlockSpec((3 * H, 2 * H), full2),
            pl.BlockSpec((1, 2 * H), full2),
            pl.BlockSpec((2 * H, H), full2),
            pl.BlockSpec((1, H), full2),
            pl.BlockSpec((H, OUT), full2),
            pl.BlockSpec((1, OUT), full2),
        ],
        out_specs=pl.BlockSpec((RB, OUT), row),
        out_shape=jax.ShapeDtypeStruct((N, OUT), jnp.float32),
    )(S, h, hg, x, u, v, mx, cnt, den_e,
      b_gat.reshape(1, H), b_gcn.reshape(1, H), W_sage_l, W_sage_r,
      b_sage.reshape(1, H), gamma_s.reshape(1, 3 * H), beta.reshape(1, 3 * H),
      W1, b1.reshape(1, 2 * H), W2, b2.reshape(1, H), W3, b3.reshape(1, OUT))


def kernel(x, edge_index, W_gat, att_src, att_dst, b_gat, W_gcn, b_gcn,
           W_sage_l, W_sage_r, b_sage, gamma, beta, W1, b1, W2, b2, W3, b3):
    npad = EP - E
    pad_src = jnp.zeros((npad,), jnp.int32)
    pad_dst = N + (jnp.arange(npad, dtype=jnp.int32) % (NPAD - N))
    src2d = jnp.concatenate([edge_index[0], pad_src]).reshape(NG, GW)
    dst2d = jnp.concatenate([edge_index[1], pad_dst]).reshape(NG, GW)

    h, hg, u, v, mx = _tc_prep(x, W_gat, W_gcn, att_src, att_dst)
    mx16 = jnp.broadcast_to(mx.reshape(1), (16,))

    ex2d, den_part, cnt_part = _sc_scalar(src2d, dst2d,
                                          u.reshape(N), v.reshape(N), mx16)
    G, cnt, den_e = _tc_mid(h, hg, x, cnt_part.T[:N], den_part.T[:N])
    S = _sc_feat(G, src2d, dst2d, ex2d)

    gamma_s = gamma * (1.0 / jnp.sqrt(1.0 + 1e-5))
    return _tc_final(S, h, hg, x, u, v, mx, cnt, den_e,
                     b_gat, b_gcn, W_sage_l, W_sage_r, b_sage, gamma_s, beta,
                     W1, b1, W2, b2, W3, b3)


# speedup vs baseline: 11.9147x; 1.0951x over previous
import dataclasses
import functools

import jax
import jax.numpy as jnp
from jax import lax
from jax.experimental import pallas as pl
from jax.experimental.pallas import tpu as pltpu
from jax.experimental.pallas import tpu_sc as plsc

N = 10000
E = 320000
D = 128
H = 128
OUT = 128

NC = 2
NS = 16
GW = 128
NG = 2560
EP = NG * GW
GPT = NG // (NC * NS)
NPAD = 10240
RB = 1000
HC = 64


def _sc_compiler_params():
    cp = pltpu.CompilerParams()
    if "needs_layout_passes" in pltpu.CompilerParams.__dataclass_fields__:
        cp = dataclasses.replace(cp, needs_layout_passes=False)
    return cp


def _prep_body(x_ref, wg_ref, wc_ref, as_ref, ad_ref,
               h_ref, hg_ref, u_ref, v_ref, mx_ref):
    xb = x_ref[...]
    hb = jnp.dot(xb, wg_ref[...], preferred_element_type=jnp.float32)
    h_ref[...] = hb
    hg_ref[...] = jnp.dot(xb, wc_ref[...], preferred_element_type=jnp.float32)
    ub = jnp.sum(hb * as_ref[...], axis=1)
    vb = jnp.sum(hb * ad_ref[...], axis=1)
    u_ref[...] = ub[:, None]
    v_ref[...] = vb[:, None]

    @pl.when(pl.program_id(0) == 0)
    def _():
        mx_ref[...] = jnp.full((1, 1), -1e30, jnp.float32)

    mx_ref[...] = jnp.maximum(mx_ref[...], jnp.max(ub))


def _tc_prep(x, W_gat, W_gcn, att_src, att_dst):
    return pl.pallas_call(
        _prep_body,
        grid=(N // RB,),
        in_specs=[
            pl.BlockSpec((RB, D), lambda i: (i, 0)),
            pl.BlockSpec((D, H), lambda i: (0, 0)),
            pl.BlockSpec((D, H), lambda i: (0, 0)),
            pl.BlockSpec((1, H), lambda i: (0, 0)),
            pl.BlockSpec((1, H), lambda i: (0, 0)),
        ],
        out_specs=[
            pl.BlockSpec((RB, H), lambda i: (i, 0)),
            pl.BlockSpec((RB, H), lambda i: (i, 0)),
            pl.BlockSpec((RB, 1), lambda i: (i, 0)),
            pl.BlockSpec((RB, 1), lambda i: (i, 0)),
            pl.BlockSpec((1, 1), lambda i: (0, 0)),
        ],
        out_shape=[
            jax.ShapeDtypeStruct((N, H), jnp.float32),
            jax.ShapeDtypeStruct((N, H), jnp.float32),
            jax.ShapeDtypeStruct((N, 1), jnp.float32),
            jax.ShapeDtypeStruct((N, 1), jnp.float32),
            jax.ShapeDtypeStruct((1, 1), jnp.float32),
        ],
    )(x, W_gat, W_gcn, att_src.reshape(1, H), att_dst.reshape(1, H))


def _sc_scalar_body(src_hbm, dst_hbm, u_hbm, v_hbm, mx_hbm,
                    ex_hbm, den_hbm, cnt_hbm,
                    u_tab, v_tab, mx_tab, sbuf, dbuf, exbuf, ones, zbuf,
                    den_sp, cnt_sp):
    cid = lax.axis_index("c")
    sid = lax.axis_index("s")
    wid = sid * NC + cid

    @pl.loop(0, 128)
    def _(i):
        zbuf[pl.ds(i * 16, 16)] = jnp.zeros((16,), jnp.float32)

    @pl.when(sid == 0)
    def _():
        @pl.loop(0, 5)
        def _(k):
            pltpu.sync_copy(zbuf, den_sp.at[pl.ds(k * 2048, 2048)])
            pltpu.sync_copy(zbuf, cnt_sp.at[pl.ds(k * 2048, 2048)])

    @pl.loop(0, GW // 16)
    def _(i):
        ones[pl.ds(i * 16, 16)] = jnp.full((16,), 1.0, jnp.float32)

    pltpu.sync_copy(u_hbm, u_tab)
    pltpu.sync_copy(v_hbm, v_tab)
    pltpu.sync_copy(mx_hbm, mx_tab)
    plsc.subcore_barrier()

    mx = mx_tab[...]
    row0 = wid * GPT

    @pl.loop(0, 10)
    def _(c):
        r = row0 + c * 8
        pltpu.sync_copy(src_hbm.at[pl.ds(r, 8)], sbuf)
        pltpu.sync_copy(dst_hbm.at[pl.ds(r, 8)], dbuf)

        @pl.loop(0, 8)
        def _(j):
            @pl.loop(0, GW // 16)
            def _(i):
                s16 = sbuf[j, pl.ds(i * 16, 16)]
                d16 = dbuf[j, pl.ds(i * 16, 16)]
                ug = plsc.load_gather(u_tab, [s16])
                vg = plsc.load_gather(v_tab, [d16])
                e = ug + vg
                e = jnp.where(e > 0, e, 0.2 * e)
                mp = mx + vg
                mp = jnp.where(mp > 0, mp, 0.2 * mp)
                exbuf[j, pl.ds(i * 16, 16)] = jnp.exp(e - mp)

        pltpu.sync_copy(exbuf, ex_hbm.at[pl.ds(r, 8)])

        @pl.loop(0, 8)
        def _(j):
            pltpu.sync_copy(exbuf.at[j], den_sp.at[dbuf.at[j]], add=True)
            pltpu.sync_copy(ones, cnt_sp.at[dbuf.at[j]], add=True)

    plsc.subcore_barrier()

    @pl.when(sid == 0)
    def _():
        pltpu.sync_copy(den_sp.at[pl.ds(0, NPAD)], den_hbm.at[cid])
        pltpu.sync_copy(cnt_sp.at[pl.ds(0, NPAD)], cnt_hbm.at[cid])


def _sc_scalar(src2d, dst2d, u, v, mx16):
    mesh = plsc.VectorSubcoreMesh(core_axis_name="c", subcore_axis_name="s")
    kern = pl.kernel(
        _sc_scalar_body,
        mesh=mesh,
        out_type=(
            jax.ShapeDtypeStruct((NG, GW), jnp.float32),
            jax.ShapeDtypeStruct((NC, NPAD), jnp.float32),
            jax.ShapeDtypeStruct((NC, NPAD), jnp.float32),
        ),
        scratch_types=[
            pltpu.VMEM((N,), jnp.float32),
            pltpu.VMEM((N,), jnp.float32),
            pltpu.VMEM((16,), jnp.float32),
            pltpu.VMEM((8, GW), jnp.int32),
            pltpu.VMEM((8, GW), jnp.int32),
            pltpu.VMEM((8, GW), jnp.float32),
            pltpu.VMEM((GW,), jnp.float32),
            pltpu.VMEM((2048,), jnp.float32),
            pltpu.VMEM_SHARED((NPAD,), jnp.float32),
            pltpu.VMEM_SHARED((NPAD,), jnp.float32),
        ],
        compiler_params=_sc_compiler_params(),
    )
    return kern(src2d, dst2d, u, v, mx16)


def _mid_body(h_ref, hg_ref, x_ref, cp_ref, dp_ref,
              g_ref, cnt_ref, den_ref):
    cnt = cp_ref[...][:, 0] + cp_ref[...][:, 1]
    den = dp_ref[...][:, 0] + dp_ref[...][:, 1]
    cnt_ref[...] = cnt[:, None]
    den_ref[...] = den[:, None]
    dinv = lax.rsqrt(cnt + 1.0)
    g_ref[0] = h_ref[...]
    g_ref[1] = hg_ref[...] * dinv[:, None]
    g_ref[2] = x_ref[...]


def _tc_mid(h, hg, x, cnt_part, den_part):
    return pl.pallas_call(
        _mid_body,
        grid=(N // RB,),
        in_specs=[
            pl.BlockSpec((RB, H), lambda i: (i, 0)),
            pl.BlockSpec((RB, H), lambda i: (i, 0)),
            pl.BlockSpec((RB, D), lambda i: (i, 0)),
            pl.BlockSpec((RB, NC), lambda i: (i, 0)),
            pl.BlockSpec((RB, NC), lambda i: (i, 0)),
        ],
        out_specs=[
            pl.BlockSpec((3, RB, H), lambda i: (0, i, 0)),
            pl.BlockSpec((RB, 1), lambda i: (i, 0)),
            pl.BlockSpec((RB, 1), lambda i: (i, 0)),
        ],
        out_shape=[
            jax.ShapeDtypeStruct((3, N, H), jnp.float32),
            jax.ShapeDtypeStruct((N, 1), jnp.float32),
            jax.ShapeDtypeStruct((N, 1), jnp.float32),
        ],
    )(h, hg, x, cnt_part, den_part)


def _sc_feat_body(g_hbm, src_hbm, dst_hbm, ex_hbm, z_hbm,
                  s_hbm,
                  rows0, rows1, sbuf, dbuf, exbuf, acc,
                  gsem0, gsem1, ssem0, ssem1):
    cid = lax.axis_index("c")
    sid = lax.axis_index("s")
    wid = sid * NC + cid
    row0 = wid * GPT

    def mul_rows(rows, g):
        gb = jnp.broadcast_to(g, (16,))

        @pl.loop(0, GW)
        def _(t):
            exv = plsc.load_gather(exbuf, [gb, jnp.broadcast_to(t, (16,))])
            for k in range(H // 16):
                rows[t, pl.ds(k * 16, 16)] = rows[t, pl.ds(k * 16, 16)] * exv

    for f in range(3):
        gq = g_hbm.at[f]
        sq = s_hbm.at[NC * f + cid]

        @pl.loop(0, 5)
        def _(k):
            pltpu.sync_copy(z_hbm, acc.at[pl.ds(sid * 640 + k * 128, 128)])

        plsc.subcore_barrier()

        @pl.loop(0, 10)
        def _(c):
            r = row0 + c * 8
            pltpu.sync_copy(src_hbm.at[pl.ds(r, 8)], sbuf)
            pltpu.sync_copy(dst_hbm.at[pl.ds(r, 8)], dbuf)
            if f == 0:
                pltpu.sync_copy(ex_hbm.at[pl.ds(r, 8)], exbuf)
            pltpu.async_copy(gq.at[sbuf.at[0]], rows0, gsem0)

            @pl.loop(0, 4)
            def _(p):
                e = 2 * p
                o = e + 1
                pltpu.make_async_copy(gq.at[sbuf.at[e]], rows0, gsem0).wait()

                @pl.when(p > 0)
                def _():
                    pltpu.make_async_copy(rows1, acc.at[dbuf.at[e - 1]],
                                          ssem1).wait()

                pltpu.async_copy(gq.at[sbuf.at[o]], rows1, gsem1)
                if f == 0:
                    mul_rows(rows0, e)
                pltpu.async_copy(rows0, acc.at[dbuf.at[e]], ssem0, add=True)
                pltpu.make_async_copy(gq.at[sbuf.at[o]], rows1, gsem1).wait()
                pltpu.make_async_copy(rows0, acc.at[dbuf.at[e]], ssem0).wait()

                @pl.when(p < 3)
                def _():
                    pltpu.async_copy(gq.at[sbuf.at[e + 2]], rows0, gsem0)

                if f == 0:
                    mul_rows(rows1, o)
                pltpu.async_copy(rows1, acc.at[dbuf.at[o]], ssem1, add=True)

            pltpu.make_async_copy(rows1, acc.at[dbuf.at[7]], ssem1).wait()

        plsc.subcore_barrier()
        pltpu.sync_copy(acc.at[pl.ds(sid * 640, 640)],
                        sq.at[pl.ds(sid * 640, 640)])
        plsc.subcore_barrier()


def _sc_feat(G, src2d, dst2d, ex2d, zeros):
    mesh = plsc.VectorSubcoreMesh(core_axis_name="c", subcore_axis_name="s")
    kern = pl.kernel(
        _sc_feat_body,
        mesh=mesh,
        out_type=jax.ShapeDtypeStruct((3 * NC, NPAD, H), jnp.float32),
        scratch_types=[
            pltpu.VMEM((GW, H), jnp.float32),
            pltpu.VMEM((GW, H), jnp.float32),
            pltpu.VMEM((8, GW), jnp.int32),
            pltpu.VMEM((8, GW), jnp.int32),
            pltpu.VMEM((8, GW), jnp.float32),
            pltpu.VMEM_SHARED((NPAD, H), jnp.float32),
            pltpu.SemaphoreType.DMA,
            pltpu.SemaphoreType.DMA,
            pltpu.SemaphoreType.DMA,
            pltpu.SemaphoreType.DMA,
        ],
        compiler_params=_sc_compiler_params(),
    )
    return kern(G, src2d, dst2d, ex2d, zeros)


def _final_body(s_ref, h_ref, hg_ref, x_ref, u_ref, v_ref, mx_ref,
                cnt_ref, den_ref,
                bgat_ref, bgcn_ref, wsl_ref, wsr_ref, bsage_ref,
                gamma_ref, beta_ref, w1_ref, b1_ref, w2_ref, b2_ref,
                w3_ref, b3_ref, out_ref):
    sgat = s_ref[0] + s_ref[1]
    sgcn = s_ref[2] + s_ref[3]
    ssag = s_ref[4] + s_ref[5]

    u = u_ref[...][:, 0]
    v = v_ref[...][:, 0]
    mx = mx_ref[...][0, 0]
    es = u + v
    es = jnp.where(es > 0, es, 0.2 * es)
    mp = mx + v
    mp = jnp.where(mp > 0, mp, 0.2 * mp)
    exs = jnp.exp(es - mp)

    den = den_ref[...][:, 0] + exs
    r = 1.0 / (den + 1e-16)
    x_gat = (sgat + exs[:, None] * h_ref[...]) * r[:, None] + bgat_ref[...]

    cnt = cnt_ref[...][:, 0]
    dinv = lax.rsqrt(cnt + 1.0)
    x_gcn = (dinv[:, None] * sgcn + (dinv * dinv)[:, None] * hg_ref[...]
             + bgcn_ref[...])

    agg = ssag / jnp.maximum(cnt, 1.0)[:, None]
    x_sage = (jnp.dot(agg, wsl_ref[...], preferred_element_type=jnp.float32)
              + bsage_ref[...]
              + jnp.dot(x_ref[...], wsr_ref[...],
                        preferred_element_type=jnp.float32))

    cat = jnp.concatenate([x_gat, x_gcn, x_sage], axis=1)
    bn = cat * gamma_ref[...] + beta_ref[...]
    h1 = jnp.dot(jnp.maximum(bn, 0.0), w1_ref[...],
                 preferred_element_type=jnp.float32) + b1_ref[...]
    h2 = jnp.dot(jnp.maximum(h1, 0.0), w2_ref[...],
                 preferred_element_type=jnp.float32) + b2_ref[...]
    out_ref[...] = jnp.dot(jnp.maximum(h2, 0.0), w3_ref[...],
                           preferred_element_type=jnp.float32) + b3_ref[...]


def _tc_final(S, h, hg, x, u, v, mx, cnt, den_e,
              b_gat, b_gcn, W_sage_l, W_sage_r, b_sage, gamma_s, beta,
              W1, b1, W2, b2, W3, b3):
    row = lambda i: (i, 0)
    full2 = lambda i: (0, 0)
    return pl.pallas_call(
        _final_body,
        grid=(N // RB,),
        in_specs=[
            pl.BlockSpec((3 * NC, RB, H), lambda i: (0, i, 0)),
            pl.BlockSpec((RB, H), row),
            pl.BlockSpec((RB, H), row),
            pl.BlockSpec((RB, D), row),
            pl.BlockSpec((RB, 1), row),
            pl.BlockSpec((RB, 1), row),
            pl.BlockSpec((1, 1), full2),
            pl.BlockSpec((RB, 1), row),
            pl.BlockSpec((RB, 1), row),
            pl.BlockSpec((1, H), full2),
            pl.BlockSpec((1, H), full2),
            pl.BlockSpec((D, H), full2),
            pl.BlockSpec((D, H), full2),
            pl.BlockSpec((1, H), full2),
            pl.BlockSpec((1, 3 * H), full2),
            pl.BlockSpec((1, 3 * H), full2),
            pl.BlockSpec((3 * H, 2 * H), full2),
            pl.BlockSpec((1, 2 * H), full2),
            pl.BlockSpec((2 * H, H), full2),
            pl.BlockSpec((1, H), full2),
            pl.BlockSpec((H, OUT), full2),
            pl.BlockSpec((1, OUT), full2),
        ],
        out_specs=pl.BlockSpec((RB, OUT), row),
        out_shape=jax.ShapeDtypeStruct((N, OUT), jnp.float32),
    )(S, h, hg, x, u, v, mx, cnt, den_e,
      b_gat.reshape(1, H), b_gcn.reshape(1, H), W_sage_l, W_sage_r,
      b_sage.reshape(1, H), gamma_s.reshape(1, 3 * H), beta.reshape(1, 3 * H),
      W1, b1.reshape(1, 2 * H), W2, b2.reshape(1, H), W3, b3.reshape(1, OUT))


def kernel(x, edge_index, W_gat, att_src, att_dst, b_gat, W_gcn, b_gcn,
           W_sage_l, W_sage_r, b_sage, gamma, beta, W1, b1, W2, b2, W3, b3):
    npad = EP - E
    pad_src = jnp.zeros((npad,), jnp.int32)
    pad_dst = N + (jnp.arange(npad, dtype=jnp.int32) % (NPAD - N))
    src2d = jnp.concatenate([edge_index[0], pad_src]).reshape(NG, GW)
    dst2d = jnp.concatenate([edge_index[1], pad_dst]).reshape(NG, GW)

    h, hg, u, v, mx = _tc_prep(x, W_gat, W_gcn, att_src, att_dst)
    mx16 = jnp.broadcast_to(mx.reshape(1), (16,))

    ex2d, den_part, cnt_part = _sc_scalar(src2d, dst2d,
                                          u.reshape(N), v.reshape(N), mx16)
    G, cnt, den_e = _tc_mid(h, hg, x, cnt_part.T[:N], den_part.T[:N])
    S = _sc_feat(G, src2d, dst2d, ex2d, jnp.zeros((128, H), jnp.float32))

    gamma_s = gamma * (1.0 / jnp.sqrt(1.0 + 1e-5))
    return _tc_final(S, h, hg, x, u, v, mx, cnt, den_e,
                     b_gat, b_gcn, W_sage_l, W_sage_r, b_sage, gamma_s, beta,
                     W1, b1, W2, b2, W3, b3)


# speedup vs baseline: 28.8784x; 2.4238x over previous
import dataclasses
import functools

import jax
import jax.numpy as jnp
from jax import lax
from jax.experimental import pallas as pl
from jax.experimental.pallas import tpu as pltpu
from jax.experimental.pallas import tpu_sc as plsc

N = 10000
E = 320000
D = 128
H = 128
OUT = 128

NC = 2
NS = 16
GW = 128
NG = 2560
EP = NG * GW
GPT = NG // (NC * NS)
NPAD = 10240
RB = 1000
HC = 64


def _sc_compiler_params():
    cp = pltpu.CompilerParams()
    if "needs_layout_passes" in pltpu.CompilerParams.__dataclass_fields__:
        cp = dataclasses.replace(cp, needs_layout_passes=False)
    return cp


def _prep_body(x_ref, wg_ref, wc_ref, as_ref, ad_ref,
               h_ref, hg_ref, u_ref, v_ref, mx_ref):
    xb = x_ref[...]
    hb = jnp.dot(xb, wg_ref[...], preferred_element_type=jnp.float32)
    h_ref[...] = hb
    hg_ref[...] = jnp.dot(xb, wc_ref[...], preferred_element_type=jnp.float32)
    ub = jnp.sum(hb * as_ref[...], axis=1)
    vb = jnp.sum(hb * ad_ref[...], axis=1)
    u_ref[...] = ub[:, None]
    v_ref[...] = vb[:, None]

    @pl.when(pl.program_id(0) == 0)
    def _():
        mx_ref[...] = jnp.full((1, 1), -1e30, jnp.float32)

    mx_ref[...] = jnp.maximum(mx_ref[...], jnp.max(ub))


def _tc_prep(x, W_gat, W_gcn, att_src, att_dst):
    return pl.pallas_call(
        _prep_body,
        grid=(N // RB,),
        in_specs=[
            pl.BlockSpec((RB, D), lambda i: (i, 0)),
            pl.BlockSpec((D, H), lambda i: (0, 0)),
            pl.BlockSpec((D, H), lambda i: (0, 0)),
            pl.BlockSpec((1, H), lambda i: (0, 0)),
            pl.BlockSpec((1, H), lambda i: (0, 0)),
        ],
        out_specs=[
            pl.BlockSpec((RB, H), lambda i: (i, 0)),
            pl.BlockSpec((RB, H), lambda i: (i, 0)),
            pl.BlockSpec((RB, 1), lambda i: (i, 0)),
            pl.BlockSpec((RB, 1), lambda i: (i, 0)),
            pl.BlockSpec((1, 1), lambda i: (0, 0)),
        ],
        out_shape=[
            jax.ShapeDtypeStruct((N, H), jnp.float32),
            jax.ShapeDtypeStruct((N, H), jnp.float32),
            jax.ShapeDtypeStruct((N, 1), jnp.float32),
            jax.ShapeDtypeStruct((N, 1), jnp.float32),
            jax.ShapeDtypeStruct((1, 1), jnp.float32),
        ],
    )(x, W_gat, W_gcn, att_src.reshape(1, H), att_dst.reshape(1, H))


def _sc_scalar_body(src_hbm, dst_hbm, u_hbm, v_hbm, mx_hbm,
                    ex_hbm, den_hbm, cnt_hbm,
                    u_tab, v_tab, mx_tab, sbuf, dbuf, exbuf, ones, zbuf,
                    den_sp, cnt_sp):
    cid = lax.axis_index("c")
    sid = lax.axis_index("s")
    wid = sid * NC + cid

    @pl.loop(0, 128)
    def _(i):
        zbuf[pl.ds(i * 16, 16)] = jnp.zeros((16,), jnp.float32)

    @pl.when(sid == 0)
    def _():
        @pl.loop(0, 5)
        def _(k):
            pltpu.sync_copy(zbuf, den_sp.at[pl.ds(k * 2048, 2048)])
            pltpu.sync_copy(zbuf, cnt_sp.at[pl.ds(k * 2048, 2048)])

    @pl.loop(0, GW // 16)
    def _(i):
        ones[pl.ds(i * 16, 16)] = jnp.full((16,), 1.0, jnp.float32)

    pltpu.sync_copy(u_hbm, u_tab)
    pltpu.sync_copy(v_hbm, v_tab)
    pltpu.sync_copy(mx_hbm, mx_tab)
    plsc.subcore_barrier()

    mx = mx_tab[...]
    row0 = wid * GPT

    @pl.loop(0, 10)
    def _(c):
        r = row0 + c * 8
        pltpu.sync_copy(src_hbm.at[pl.ds(r, 8)], sbuf)
        pltpu.sync_copy(dst_hbm.at[pl.ds(r, 8)], dbuf)

        @pl.loop(0, 8)
        def _(j):
            @pl.loop(0, GW // 16)
            def _(i):
                s16 = sbuf[j, pl.ds(i * 16, 16)]
                d16 = dbuf[j, pl.ds(i * 16, 16)]
                ug = plsc.load_gather(u_tab, [s16])
                vg = plsc.load_gather(v_tab, [d16])
                e = ug + vg
                e = jnp.where(e > 0, e, 0.2 * e)
                mp = mx + vg
                mp = jnp.where(mp > 0, mp, 0.2 * mp)
                exbuf[j, pl.ds(i * 16, 16)] = jnp.exp(e - mp)

        pltpu.sync_copy(exbuf, ex_hbm.at[pl.ds(r, 8)])

        @pl.loop(0, 8)
        def _(j):
            pltpu.sync_copy(exbuf.at[j], den_sp.at[dbuf.at[j]], add=True)
            pltpu.sync_copy(ones, cnt_sp.at[dbuf.at[j]], add=True)

    plsc.subcore_barrier()

    @pl.when(sid == 0)
    def _():
        pltpu.sync_copy(den_sp.at[pl.ds(0, NPAD)], den_hbm.at[cid])
        pltpu.sync_copy(cnt_sp.at[pl.ds(0, NPAD)], cnt_hbm.at[cid])


def _sc_scalar(src2d, dst2d, u, v, mx16):
    mesh = plsc.VectorSubcoreMesh(core_axis_name="c", subcore_axis_name="s")
    kern = pl.kernel(
        _sc_scalar_body,
        mesh=mesh,
        out_type=(
            jax.ShapeDtypeStruct((NG, GW), jnp.float32),
            jax.ShapeDtypeStruct((NC, NPAD), jnp.float32),
            jax.ShapeDtypeStruct((NC, NPAD), jnp.float32),
        ),
        scratch_types=[
            pltpu.VMEM((NPAD,), jnp.float32),
            pltpu.VMEM((NPAD,), jnp.float32),
            pltpu.VMEM((16,), jnp.float32),
            pltpu.VMEM((8, GW), jnp.int32),
            pltpu.VMEM((8, GW), jnp.int32),
            pltpu.VMEM((8, GW), jnp.float32),
            pltpu.VMEM((GW,), jnp.float32),
            pltpu.VMEM((2048,), jnp.float32),
            pltpu.VMEM_SHARED((NPAD,), jnp.float32),
            pltpu.VMEM_SHARED((NPAD,), jnp.float32),
        ],
        compiler_params=_sc_compiler_params(),
    )
    return kern(src2d, dst2d, u, v, mx16)


def _mid_body(h_ref, hg_ref, x_ref, cp_ref, dp_ref,
              g_ref, cnt_ref, den_ref):
    cnt = cp_ref[...][:, 0] + cp_ref[...][:, 1]
    den = dp_ref[...][:, 0] + dp_ref[...][:, 1]
    cnt_ref[...] = cnt[:, None]
    den_ref[...] = den[:, None]
    dinv = lax.rsqrt(cnt + 1.0)
    g_ref[0] = h_ref[...]
    g_ref[1] = hg_ref[...] * dinv[:, None]
    g_ref[2] = x_ref[...]


def _tc_mid(h, hg, x, cnt_part, den_part):
    return pl.pallas_call(
        _mid_body,
        grid=(N // RB,),
        in_specs=[
            pl.BlockSpec((RB, H), lambda i: (i, 0)),
            pl.BlockSpec((RB, H), lambda i: (i, 0)),
            pl.BlockSpec((RB, D), lambda i: (i, 0)),
            pl.BlockSpec((RB, NC), lambda i: (i, 0)),
            pl.BlockSpec((RB, NC), lambda i: (i, 0)),
        ],
        out_specs=[
            pl.BlockSpec((3, RB, H), lambda i: (0, i, 0)),
            pl.BlockSpec((RB, 1), lambda i: (i, 0)),
            pl.BlockSpec((RB, 1), lambda i: (i, 0)),
        ],
        out_shape=[
            jax.ShapeDtypeStruct((3, N, H), jnp.float32),
            jax.ShapeDtypeStruct((N, 1), jnp.float32),
            jax.ShapeDtypeStruct((N, 1), jnp.float32),
        ],
    )(h, hg, x, cnt_part, den_part)


def _sc_feat_body(g_hbm, src_hbm, dst_hbm, ex_hbm, z_hbm,
                  s_hbm,
                  rows0, rows1, sbuf, dbuf, exbuf, acc,
                  gsem0, gsem1, ssem0, ssem1):
    cid = lax.axis_index("c")
    sid = lax.axis_index("s")
    wid = sid * NC + cid
    row0 = wid * GPT

    def mul_rows(rows, g):
        gb = jnp.broadcast_to(g, (16,))

        @pl.loop(0, GW)
        def _(t):
            exv = plsc.load_gather(exbuf, [gb, jnp.broadcast_to(t, (16,))])
            for k in range(H // 16):
                rows[t, pl.ds(k * 16, 16)] = rows[t, pl.ds(k * 16, 16)] * exv

    for f in range(3):
        gq = g_hbm.at[f]
        sq = s_hbm.at[NC * f + cid]

        @pl.loop(0, 5)
        def _(k):
            pltpu.sync_copy(z_hbm, acc.at[pl.ds(sid * 640 + k * 128, 128)])

        plsc.subcore_barrier()

        @pl.loop(0, 10)
        def _(c):
            r = row0 + c * 8
            pltpu.sync_copy(src_hbm.at[pl.ds(r, 8)], sbuf)
            pltpu.sync_copy(dst_hbm.at[pl.ds(r, 8)], dbuf)
            if f == 0:
                pltpu.sync_copy(ex_hbm.at[pl.ds(r, 8)], exbuf)
            pltpu.async_copy(gq.at[sbuf.at[0]], rows0, gsem0)

            @pl.loop(0, 4)
            def _(p):
                e = 2 * p
                o = e + 1
                pltpu.make_async_copy(gq.at[sbuf.at[e]], rows0, gsem0).wait()

                @pl.when(p > 0)
                def _():
                    pltpu.make_async_copy(rows1, acc.at[dbuf.at[e - 1]],
                                          ssem1).wait()

                pltpu.async_copy(gq.at[sbuf.at[o]], rows1, gsem1)
                if f == 0:
                    mul_rows(rows0, e)
                pltpu.async_copy(rows0, acc.at[dbuf.at[e]], ssem0, add=True)
                pltpu.make_async_copy(gq.at[sbuf.at[o]], rows1, gsem1).wait()
                pltpu.make_async_copy(rows0, acc.at[dbuf.at[e]], ssem0).wait()

                @pl.when(p < 3)
                def _():
                    pltpu.async_copy(gq.at[sbuf.at[e + 2]], rows0, gsem0)

                if f == 0:
                    mul_rows(rows1, o)
                pltpu.async_copy(rows1, acc.at[dbuf.at[o]], ssem1, add=True)

            pltpu.make_async_copy(rows1, acc.at[dbuf.at[7]], ssem1).wait()

        plsc.subcore_barrier()
        pltpu.sync_copy(acc.at[pl.ds(sid * 640, 640)],
                        sq.at[pl.ds(sid * 640, 640)])
        plsc.subcore_barrier()


def _sc_feat(G, src2d, dst2d, ex2d, zeros):
    mesh = plsc.VectorSubcoreMesh(core_axis_name="c", subcore_axis_name="s")
    kern = pl.kernel(
        _sc_feat_body,
        mesh=mesh,
        out_type=jax.ShapeDtypeStruct((3 * NC, NPAD, H), jnp.float32),
        scratch_types=[
            pltpu.VMEM((GW, H), jnp.float32),
            pltpu.VMEM((GW, H), jnp.float32),
            pltpu.VMEM((8, GW), jnp.int32),
            pltpu.VMEM((8, GW), jnp.int32),
            pltpu.VMEM((8, GW), jnp.float32),
            pltpu.VMEM_SHARED((NPAD, H), jnp.float32),
            pltpu.SemaphoreType.DMA,
            pltpu.SemaphoreType.DMA,
            pltpu.SemaphoreType.DMA,
            pltpu.SemaphoreType.DMA,
        ],
        compiler_params=_sc_compiler_params(),
    )
    return kern(G, src2d, dst2d, ex2d, zeros)


def _final_body(s_ref, h_ref, hg_ref, x_ref, u_ref, v_ref, mx_ref,
                cnt_ref, den_ref,
                bgat_ref, bgcn_ref, wsl_ref, wsr_ref, bsage_ref,
                gamma_ref, beta_ref, w1_ref, b1_ref, w2_ref, b2_ref,
                w3_ref, b3_ref, out_ref):
    sgat = s_ref[0] + s_ref[1]
    sgcn = s_ref[2] + s_ref[3]
    ssag = s_ref[4] + s_ref[5]

    u = u_ref[...][:, 0]
    v = v_ref[...][:, 0]
    mx = mx_ref[...][0, 0]
    es = u + v
    es = jnp.where(es > 0, es, 0.2 * es)
    mp = mx + v
    mp = jnp.where(mp > 0, mp, 0.2 * mp)
    exs = jnp.exp(es - mp)

    den = den_ref[...][:, 0] + exs
    r = 1.0 / (den + 1e-16)
    x_gat = (sgat + exs[:, None] * h_ref[...]) * r[:, None] + bgat_ref[...]

    cnt = cnt_ref[...][:, 0]
    dinv = lax.rsqrt(cnt + 1.0)
    x_gcn = (dinv[:, None] * sgcn + (dinv * dinv)[:, None] * hg_ref[...]
             + bgcn_ref[...])

    agg = ssag / jnp.maximum(cnt, 1.0)[:, None]
    x_sage = (jnp.dot(agg, wsl_ref[...], preferred_element_type=jnp.float32)
              + bsage_ref[...]
              + jnp.dot(x_ref[...], wsr_ref[...],
                        preferred_element_type=jnp.float32))

    cat = jnp.concatenate([x_gat, x_gcn, x_sage], axis=1)
    bn = cat * gamma_ref[...] + beta_ref[...]
    h1 = jnp.dot(jnp.maximum(bn, 0.0), w1_ref[...],
                 preferred_element_type=jnp.float32) + b1_ref[...]
    h2 = jnp.dot(jnp.maximum(h1, 0.0), w2_ref[...],
                 preferred_element_type=jnp.float32) + b2_ref[...]
    out_ref[...] = jnp.dot(jnp.maximum(h2, 0.0), w3_ref[...],
                           preferred_element_type=jnp.float32) + b3_ref[...]


def _tc_final(S, h, hg, x, u, v, mx, cnt, den_e,
              b_gat, b_gcn, W_sage_l, W_sage_r, b_sage, gamma_s, beta,
              W1, b1, W2, b2, W3, b3):
    row = lambda i: (i, 0)
    full2 = lambda i: (0, 0)
    return pl.pallas_call(
        _final_body,
        grid=(N // RB,),
        in_specs=[
            pl.BlockSpec((3 * NC, RB, H), lambda i: (0, i, 0)),
            pl.BlockSpec((RB, H), row),
            pl.BlockSpec((RB, H), row),
            pl.BlockSpec((RB, D), row),
            pl.BlockSpec((RB, 1), row),
            pl.BlockSpec((RB, 1), row),
            pl.BlockSpec((1, 1), full2),
            pl.BlockSpec((RB, 1), row),
            pl.BlockSpec((RB, 1), row),
            pl.BlockSpec((1, H), full2),
            pl.BlockSpec((1, H), full2),
            pl.BlockSpec((D, H), full2),
            pl.BlockSpec((D, H), full2),
            pl.BlockSpec((1, H), full2),
            pl.BlockSpec((1, 3 * H), full2),
            pl.BlockSpec((1, 3 * H), full2),
            pl.BlockSpec((3 * H, 2 * H), full2),
            pl.BlockSpec((1, 2 * H), full2),
            pl.BlockSpec((2 * H, H), full2),
            pl.BlockSpec((1, H), full2),
            pl.BlockSpec((H, OUT), full2),
            pl.BlockSpec((1, OUT), full2),
        ],
        out_specs=pl.BlockSpec((RB, OUT), row),
        out_shape=jax.ShapeDtypeStruct((N, OUT), jnp.float32),
    )(S, h, hg, x, u, v, mx, cnt, den_e,
      b_gat.reshape(1, H), b_gcn.reshape(1, H), W_sage_l, W_sage_r,
      b_sage.reshape(1, H), gamma_s.reshape(1, 3 * H), beta.reshape(1, 3 * H),
      W1, b1.reshape(1, 2 * H), W2, b2.reshape(1, H), W3, b3.reshape(1, OUT))


def kernel(x, edge_index, W_gat, att_src, att_dst, b_gat, W_gcn, b_gcn,
           W_sage_l, W_sage_r, b_sage, gamma, beta, W1, b1, W2, b2, W3, b3):
    npad = EP - E
    pad_src = jnp.arange(npad, dtype=jnp.int32) % N
    pad_dst = N + (jnp.arange(npad, dtype=jnp.int32) % (NPAD - N))
    src2d = jnp.concatenate([edge_index[0], pad_src]).reshape(NG, GW)
    dst2d = jnp.concatenate([edge_index[1], pad_dst]).reshape(NG, GW)

    h, hg, u, v, mx = _tc_prep(x, W_gat, W_gcn, att_src, att_dst)
    mx16 = jnp.broadcast_to(mx.reshape(1), (16,))

    zpad = jnp.zeros((NPAD - N,), jnp.float32)
    ex2d, den_part, cnt_part = _sc_scalar(
        src2d, dst2d,
        jnp.concatenate([u.reshape(N), zpad]),
        jnp.concatenate([v.reshape(N), zpad]), mx16)
    G, cnt, den_e = _tc_mid(h, hg, x, cnt_part.T[:N], den_part.T[:N])
    S = _sc_feat(G, src2d, dst2d, ex2d, jnp.zeros((128, H), jnp.float32))

    gamma_s = gamma * (1.0 / jnp.sqrt(1.0 + 1e-5))
    return _tc_final(S, h, hg, x, u, v, mx, cnt, den_e,
                     b_gat, b_gcn, W_sage_l, W_sage_r, b_sage, gamma_s, beta,
                     W1, b1, W2, b2, W3, b3)


# speedup vs baseline: 29.1471x; 1.0093x over previous
import dataclasses
import functools

import jax
import jax.numpy as jnp
from jax import lax
from jax.experimental import pallas as pl
from jax.experimental.pallas import tpu as pltpu
from jax.experimental.pallas import tpu_sc as plsc

N = 10000
E = 320000
D = 128
H = 128
OUT = 128

NC = 2
NS = 16
GW = 128
NG = 2560
EP = NG * GW
GPT = NG // (NC * NS)
NPAD = 10240
RB = 1000
HC = 64


def _sc_compiler_params():
    cp = pltpu.CompilerParams()
    if "needs_layout_passes" in pltpu.CompilerParams.__dataclass_fields__:
        cp = dataclasses.replace(cp, needs_layout_passes=False)
    return cp


def _prep_body(x_ref, wg_ref, wc_ref, as_ref, ad_ref,
               h_ref, hg_ref, u_ref, v_ref, mx_ref):
    xb = x_ref[...]
    hb = jnp.dot(xb, wg_ref[...], preferred_element_type=jnp.float32)
    h_ref[...] = hb
    hg_ref[...] = jnp.dot(xb, wc_ref[...], preferred_element_type=jnp.float32)
    ub = jnp.sum(hb * as_ref[...], axis=1)
    vb = jnp.sum(hb * ad_ref[...], axis=1)
    u_ref[...] = ub[:, None]
    v_ref[...] = vb[:, None]

    @pl.when(pl.program_id(0) == 0)
    def _():
        mx_ref[...] = jnp.full((1, 1), -1e30, jnp.float32)

    mx_ref[...] = jnp.maximum(mx_ref[...], jnp.max(ub))


def _tc_prep(x, W_gat, W_gcn, att_src, att_dst):
    return pl.pallas_call(
        _prep_body,
        grid=(N // RB,),
        in_specs=[
            pl.BlockSpec((RB, D), lambda i: (i, 0)),
            pl.BlockSpec((D, H), lambda i: (0, 0)),
            pl.BlockSpec((D, H), lambda i: (0, 0)),
            pl.BlockSpec((1, H), lambda i: (0, 0)),
            pl.BlockSpec((1, H), lambda i: (0, 0)),
        ],
        out_specs=[
            pl.BlockSpec((RB, H), lambda i: (i, 0)),
            pl.BlockSpec((RB, H), lambda i: (i, 0)),
            pl.BlockSpec((RB, 1), lambda i: (i, 0)),
            pl.BlockSpec((RB, 1), lambda i: (i, 0)),
            pl.BlockSpec((1, 1), lambda i: (0, 0)),
        ],
        out_shape=[
            jax.ShapeDtypeStruct((N, H), jnp.float32),
            jax.ShapeDtypeStruct((N, H), jnp.float32),
            jax.ShapeDtypeStruct((N, 1), jnp.float32),
            jax.ShapeDtypeStruct((N, 1), jnp.float32),
            jax.ShapeDtypeStruct((1, 1), jnp.float32),
        ],
    )(x, W_gat, W_gcn, att_src.reshape(1, H), att_dst.reshape(1, H))


def _sc_scalar_body(src_hbm, dst_hbm, u_hbm, v_hbm, mx_hbm,
                    ex_hbm, den_hbm, cnt_hbm,
                    u_tab, v_tab, mx_tab, sbuf, dbuf, exbuf, ones, zbuf,
                    den_sp, cnt_sp):
    cid = lax.axis_index("c")
    sid = lax.axis_index("s")
    wid = sid * NC + cid

    @pl.loop(0, 128)
    def _(i):
        zbuf[pl.ds(i * 16, 16)] = jnp.zeros((16,), jnp.float32)

    @pl.when(sid == 0)
    def _():
        @pl.loop(0, 5)
        def _(k):
            pltpu.sync_copy(zbuf, den_sp.at[pl.ds(k * 2048, 2048)])
            pltpu.sync_copy(zbuf, cnt_sp.at[pl.ds(k * 2048, 2048)])

    @pl.loop(0, GW // 16)
    def _(i):
        ones[pl.ds(i * 16, 16)] = jnp.full((16,), 1.0, jnp.float32)

    pltpu.sync_copy(u_hbm, u_tab)
    pltpu.sync_copy(v_hbm, v_tab)
    pltpu.sync_copy(mx_hbm, mx_tab)
    plsc.subcore_barrier()

    mx = mx_tab[...]
    row0 = wid * GPT

    @pl.loop(0, 10)
    def _(c):
        r = row0 + c * 8
        pltpu.sync_copy(src_hbm.at[pl.ds(r, 8)], sbuf)
        pltpu.sync_copy(dst_hbm.at[pl.ds(r, 8)], dbuf)

        @pl.loop(0, 8)
        def _(j):
            @pl.loop(0, GW // 16)
            def _(i):
                s16 = sbuf[j, pl.ds(i * 16, 16)]
                d16 = dbuf[j, pl.ds(i * 16, 16)]
                ug = plsc.load_gather(u_tab, [s16])
                vg = plsc.load_gather(v_tab, [d16])
                e = ug + vg
                e = jnp.where(e > 0, e, 0.2 * e)
                mp = mx + vg
                mp = jnp.where(mp > 0, mp, 0.2 * mp)
                exbuf[j, pl.ds(i * 16, 16)] = jnp.exp(e - mp)

        pltpu.sync_copy(exbuf, ex_hbm.at[pl.ds(r, 8)])

        @pl.loop(0, 8)
        def _(j):
            pltpu.sync_copy(exbuf.at[j], den_sp.at[dbuf.at[j]], add=True)
            pltpu.sync_copy(ones, cnt_sp.at[dbuf.at[j]], add=True)

    plsc.subcore_barrier()

    @pl.when(sid == 0)
    def _():
        pltpu.sync_copy(den_sp.at[pl.ds(0, NPAD)], den_hbm.at[cid])
        pltpu.sync_copy(cnt_sp.at[pl.ds(0, NPAD)], cnt_hbm.at[cid])


def _sc_scalar(src2d, dst2d, u, v, mx16):
    mesh = plsc.VectorSubcoreMesh(core_axis_name="c", subcore_axis_name="s")
    kern = pl.kernel(
        _sc_scalar_body,
        mesh=mesh,
        out_type=(
            jax.ShapeDtypeStruct((NG, GW), jnp.float32),
            jax.ShapeDtypeStruct((NC, NPAD), jnp.float32),
            jax.ShapeDtypeStruct((NC, NPAD), jnp.float32),
        ),
        scratch_types=[
            pltpu.VMEM((NPAD,), jnp.float32),
            pltpu.VMEM((NPAD,), jnp.float32),
            pltpu.VMEM((16,), jnp.float32),
            pltpu.VMEM((8, GW), jnp.int32),
            pltpu.VMEM((8, GW), jnp.int32),
            pltpu.VMEM((8, GW), jnp.float32),
            pltpu.VMEM((GW,), jnp.float32),
            pltpu.VMEM((2048,), jnp.float32),
            pltpu.VMEM_SHARED((NPAD,), jnp.float32),
            pltpu.VMEM_SHARED((NPAD,), jnp.float32),
        ],
        compiler_params=_sc_compiler_params(),
    )
    return kern(src2d, dst2d, u, v, mx16)


def _mid_body(hg_ref, cp_ref, dp_ref,
              g_ref, cnt_ref, den_ref):
    cnt = cp_ref[...][:, 0] + cp_ref[...][:, 1]
    den = dp_ref[...][:, 0] + dp_ref[...][:, 1]
    cnt_ref[...] = cnt[:, None]
    den_ref[...] = den[:, None]
    dinv = lax.rsqrt(cnt + 1.0)
    g_ref[...] = hg_ref[...] * dinv[:, None]


def _tc_mid(hg, cnt_part, den_part):
    return pl.pallas_call(
        _mid_body,
        grid=(N // RB,),
        in_specs=[
            pl.BlockSpec((RB, H), lambda i: (i, 0)),
            pl.BlockSpec((RB, NC), lambda i: (i, 0)),
            pl.BlockSpec((RB, NC), lambda i: (i, 0)),
        ],
        out_specs=[
            pl.BlockSpec((RB, H), lambda i: (i, 0)),
            pl.BlockSpec((RB, 1), lambda i: (i, 0)),
            pl.BlockSpec((RB, 1), lambda i: (i, 0)),
        ],
        out_shape=[
            jax.ShapeDtypeStruct((N, H), jnp.float32),
            jax.ShapeDtypeStruct((N, 1), jnp.float32),
            jax.ShapeDtypeStruct((N, 1), jnp.float32),
        ],
    )(hg, cnt_part, den_part)


def _sc_feat_body(h_hbm, hgp_hbm, x_hbm, src_hbm, dst_hbm, ex_hbm, z_hbm,
                  s_hbm,
                  rows0, rows1, sbuf, dbuf, exbuf, acc,
                  gsem0, gsem1, ssem0, ssem1):
    cid = lax.axis_index("c")
    sid = lax.axis_index("s")
    wid = sid * NC + cid
    row0 = wid * GPT

    def mul_rows(rows, g):
        gb = jnp.broadcast_to(g, (16,))

        @pl.loop(0, GW)
        def _(t):
            exv = plsc.load_gather(exbuf, [gb, jnp.broadcast_to(t, (16,))])
            for k in range(H // 16):
                rows[t, pl.ds(k * 16, 16)] = rows[t, pl.ds(k * 16, 16)] * exv

    for f, gq in enumerate((h_hbm, hgp_hbm, x_hbm)):
        sq = s_hbm.at[NC * f + cid]

        @pl.loop(0, 5)
        def _(k):
            pltpu.sync_copy(z_hbm, acc.at[pl.ds(sid * 640 + k * 128, 128)])

        plsc.subcore_barrier()

        @pl.loop(0, 10)
        def _(c):
            r = row0 + c * 8
            pltpu.sync_copy(src_hbm.at[pl.ds(r, 8)], sbuf)
            pltpu.sync_copy(dst_hbm.at[pl.ds(r, 8)], dbuf)
            if f == 0:
                pltpu.sync_copy(ex_hbm.at[pl.ds(r, 8)], exbuf)
            pltpu.async_copy(gq.at[sbuf.at[0]], rows0, gsem0)

            @pl.loop(0, 4)
            def _(p):
                e = 2 * p
                o = e + 1
                pltpu.make_async_copy(gq.at[sbuf.at[e]], rows0, gsem0).wait()

                @pl.when(p > 0)
                def _():
                    pltpu.make_async_copy(rows1, acc.at[dbuf.at[e - 1]],
                                          ssem1).wait()

                pltpu.async_copy(gq.at[sbuf.at[o]], rows1, gsem1)
                if f == 0:
                    mul_rows(rows0, e)
                pltpu.async_copy(rows0, acc.at[dbuf.at[e]], ssem0, add=True)
                pltpu.make_async_copy(gq.at[sbuf.at[o]], rows1, gsem1).wait()
                pltpu.make_async_copy(rows0, acc.at[dbuf.at[e]], ssem0).wait()

                @pl.when(p < 3)
                def _():
                    pltpu.async_copy(gq.at[sbuf.at[e + 2]], rows0, gsem0)

                if f == 0:
                    mul_rows(rows1, o)
                pltpu.async_copy(rows1, acc.at[dbuf.at[o]], ssem1, add=True)

            pltpu.make_async_copy(rows1, acc.at[dbuf.at[7]], ssem1).wait()

        plsc.subcore_barrier()
        pltpu.sync_copy(acc.at[pl.ds(sid * 640, 640)],
                        sq.at[pl.ds(sid * 640, 640)])
        plsc.subcore_barrier()


def _sc_feat(h, hgp, x, src2d, dst2d, ex2d, zeros):
    mesh = plsc.VectorSubcoreMesh(core_axis_name="c", subcore_axis_name="s")
    kern = pl.kernel(
        _sc_feat_body,
        mesh=mesh,
        out_type=jax.ShapeDtypeStruct((3 * NC, NPAD, H), jnp.float32),
        scratch_types=[
            pltpu.VMEM((GW, H), jnp.float32),
            pltpu.VMEM((GW, H), jnp.float32),
            pltpu.VMEM((8, GW), jnp.int32),
            pltpu.VMEM((8, GW), jnp.int32),
            pltpu.VMEM((8, GW), jnp.float32),
            pltpu.VMEM_SHARED((NPAD, H), jnp.float32),
            pltpu.SemaphoreType.DMA,
            pltpu.SemaphoreType.DMA,
            pltpu.SemaphoreType.DMA,
            pltpu.SemaphoreType.DMA,
        ],
        compiler_params=_sc_compiler_params(),
    )
    return kern(h, hgp, x, src2d, dst2d, ex2d, zeros)


def _final_body(s_ref, h_ref, hg_ref, x_ref, u_ref, v_ref, mx_ref,
                cnt_ref, den_ref,
                bgat_ref, bgcn_ref, wsl_ref, wsr_ref, bsage_ref,
                gamma_ref, beta_ref, w1_ref, b1_ref, w2_ref, b2_ref,
                w3_ref, b3_ref, out_ref):
    sgat = s_ref[0] + s_ref[1]
    sgcn = s_ref[2] + s_ref[3]
    ssag = s_ref[4] + s_ref[5]

    u = u_ref[...][:, 0]
    v = v_ref[...][:, 0]
    mx = mx_ref[...][0, 0]
    es = u + v
    es = jnp.where(es > 0, es, 0.2 * es)
    mp = mx + v
    mp = jnp.where(mp > 0, mp, 0.2 * mp)
    exs = jnp.exp(es - mp)

    den = den_ref[...][:, 0] + exs
    r = 1.0 / (den + 1e-16)
    x_gat = (sgat + exs[:, None] * h_ref[...]) * r[:, None] + bgat_ref[...]

    cnt = cnt_ref[...][:, 0]
    dinv = lax.rsqrt(cnt + 1.0)
    x_gcn = (dinv[:, None] * sgcn + (dinv * dinv)[:, None] * hg_ref[...]
             + bgcn_ref[...])

    agg = ssag / jnp.maximum(cnt, 1.0)[:, None]
    x_sage = (jnp.dot(agg, wsl_ref[...], preferred_element_type=jnp.float32)
              + bsage_ref[...]
              + jnp.dot(x_ref[...], wsr_ref[...],
                        preferred_element_type=jnp.float32))

    cat = jnp.concatenate([x_gat, x_gcn, x_sage], axis=1)
    bn = cat * gamma_ref[...] + beta_ref[...]
    h1 = jnp.dot(jnp.maximum(bn, 0.0), w1_ref[...],
                 preferred_element_type=jnp.float32) + b1_ref[...]
    h2 = jnp.dot(jnp.maximum(h1, 0.0), w2_ref[...],
                 preferred_element_type=jnp.float32) + b2_ref[...]
    out_ref[...] = jnp.dot(jnp.maximum(h2, 0.0), w3_ref[...],
                           preferred_element_type=jnp.float32) + b3_ref[...]


def _tc_final(S, h, hg, x, u, v, mx, cnt, den_e,
              b_gat, b_gcn, W_sage_l, W_sage_r, b_sage, gamma_s, beta,
              W1, b1, W2, b2, W3, b3):
    row = lambda i: (i, 0)
    full2 = lambda i: (0, 0)
    return pl.pallas_call(
        _final_body,
        grid=(N // RB,),
        in_specs=[
            pl.BlockSpec((3 * NC, RB, H), lambda i: (0, i, 0)),
            pl.BlockSpec((RB, H), row),
            pl.BlockSpec((RB, H), row),
            pl.BlockSpec((RB, D), row),
            pl.BlockSpec((RB, 1), row),
            pl.BlockSpec((RB, 1), row),
            pl.BlockSpec((1, 1), full2),
            pl.BlockSpec((RB, 1), row),
            pl.BlockSpec((RB, 1), row),
            pl.BlockSpec((1, H), full2),
            pl.BlockSpec((1, H), full2),
            pl.BlockSpec((D, H), full2),
            pl.BlockSpec((D, H), full2),
            pl.BlockSpec((1, H), full2),
            pl.BlockSpec((1, 3 * H), full2),
            pl.BlockSpec((1, 3 * H), full2),
            pl.BlockSpec((3 * H, 2 * H), full2),
            pl.BlockSpec((1, 2 * H), full2),
            pl.BlockSpec((2 * H, H), full2),
            pl.BlockSpec((1, H), full2),
            pl.BlockSpec((H, OUT), full2),
            pl.BlockSpec((1, OUT), full2),
        ],
        out_specs=pl.BlockSpec((RB, OUT), row),
        out_shape=jax.ShapeDtypeStruct((N, OUT), jnp.float32),
    )(S, h, hg, x, u, v, mx, cnt, den_e,
      b_gat.reshape(1, H), b_gcn.reshape(1, H), W_sage_l, W_sage_r,
      b_sage.reshape(1, H), gamma_s.reshape(1, 3 * H), beta.reshape(1, 3 * H),
      W1, b1.reshape(1, 2 * H), W2, b2.reshape(1, H), W3, b3.reshape(1, OUT))


def kernel(x, edge_index, W_gat, att_src, att_dst, b_gat, W_gcn, b_gcn,
           W_sage_l, W_sage_r, b_sage, gamma, beta, W1, b1, W2, b2, W3, b3):
    npad = EP - E
    pad_src = jnp.arange(npad, dtype=jnp.int32) % N
    pad_dst = N + (jnp.arange(npad, dtype=jnp.int32) % (NPAD - N))
    src2d = jnp.concatenate([edge_index[0], pad_src]).reshape(NG, GW)
    dst2d = jnp.concatenate([edge_index[1], pad_dst]).reshape(NG, GW)

    h, hg, u, v, mx = _tc_prep(x, W_gat, W_gcn, att_src, att_dst)
    mx16 = jnp.broadcast_to(mx.reshape(1), (16,))

    zpad = jnp.zeros((NPAD - N,), jnp.float32)
    ex2d, den_part, cnt_part = _sc_scalar(
        src2d, dst2d,
        jnp.concatenate([u.reshape(N), zpad]),
        jnp.concatenate([v.reshape(N), zpad]), mx16)
    hgp, cnt, den_e = _tc_mid(hg, cnt_part.T[:N], den_part.T[:N])
    S = _sc_feat(h, hgp, x, src2d, dst2d, ex2d,
                 jnp.zeros((128, H), jnp.float32))

    gamma_s = gamma * (1.0 / jnp.sqrt(1.0 + 1e-5))
    return _tc_final(S, h, hg, x, u, v, mx, cnt, den_e,
                     b_gat, b_gcn, W_sage_l, W_sage_r, b_sage, gamma_s, beta,
                     W1, b1, W2, b2, W3, b3)
